# baseline (speedup 1.0000x reference)
import functools

import jax
import jax.numpy as jnp
from jax import lax
from jax.experimental import pallas as pl
from jax.experimental.pallas import tpu as pltpu
from jax.experimental.pallas import tpu_sc as plsc

NC = 2
NS = 16
NW = NC * NS
CB = 40
CB3 = 16

_LOG1P_C = (
    3.3869654e-08, 9.9999428e-01, -4.9983856e-01, 3.3154863e-01,
    -2.3982616e-01, 1.6582276e-01, -9.3252040e-02, 3.4849711e-02,
    -6.1514708e-03,
)


def _log1p_poly(t):
    y = jnp.full_like(t, _LOG1P_C[-1])
    for c in _LOG1P_C[-2::-1]:
        y = y * t + c
    return y


def _log1p_estrin(t):
    c = _LOG1P_C
    t2 = t * t
    t4 = t2 * t2
    p01 = c[0] + c[1] * t
    p23 = c[2] + c[3] * t
    p45 = c[4] + c[5] * t
    p67 = c[6] + c[7] * t
    q0 = p01 + p23 * t2
    q1 = p45 + p67 * t2
    return q0 + (q1 + c[8] * t4) * t4


def _proj_body(node_ref, ws_ref, bs_ref, wd_ref, bd_ref, hs_ref, hd_ref):
    x = node_ref[...]
    hs_ref[...] = jnp.dot(x, ws_ref[...], preferred_element_type=jnp.float32) + bs_ref[...]
    hd_ref[...] = jnp.dot(x, wd_ref[...], preferred_element_type=jnp.float32) + bd_ref[...]


def _eproj_body(ef_ref, we_ref, be_ref, out_ref):
    out_ref[...] = (
        jnp.dot(ef_ref[...], we_ref[...], preferred_element_type=jnp.float32)
        + be_ref[...]
    )


def _s2_body(epw, nchunk, hsrc, hdst, eproj, src, dst,
             m_out, stats_out,
             si0, si1, di0, di1, a0, a1, b0, b1, c0, c1, m0, m1, acc_v,
             semg0, semg1, semw0, semw1, semi0, semi1):
    cid = lax.axis_index("c")
    sid = lax.axis_index("s")
    wid = sid * NC + cid
    SI = (si0, si1)
    DI = (di0, di1)
    A = (a0, a1)
    B = (b0, b1)
    C = (c0, c1)
    M = (m0, m1)
    SG = (semg0, semg1)
    SW = (semw0, semw1)
    SEMI = (semi0, semi1)

    for r in range(32):
        acc_v[r] = jnp.zeros((16,), jnp.float32)

    def launch_i(k, b):
        base = wid * epw + k * CB
        pltpu.async_copy(src.at[pl.ds(base, CB)], SI[b], SEMI[b])
        pltpu.async_copy(dst.at[pl.ds(base, CB)], DI[b], SEMI[b])

    def wait_i(k, b):
        base = wid * epw + k * CB
        pltpu.make_async_copy(src.at[pl.ds(base, CB)], SI[b], SEMI[b]).wait()
        pltpu.make_async_copy(dst.at[pl.ds(base, CB)], DI[b], SEMI[b]).wait()

    def launch_g(k, bn):
        base = wid * epw + k * CB
        pltpu.async_copy(hsrc.at[SI[bn]], A[bn], SG[bn])
        pltpu.async_copy(hdst.at[DI[bn]], B[bn], SG[bn])
        pltpu.async_copy(eproj.at[pl.ds(base, CB)], C[bn], SG[bn])

    def wait_g(k, bn):
        base = wid * epw + k * CB
        pltpu.make_async_copy(hsrc.at[SI[bn]], A[bn], SG[bn]).wait()
        pltpu.make_async_copy(hdst.at[DI[bn]], B[bn], SG[bn]).wait()
        pltpu.make_async_copy(eproj.at[pl.ds(base, CB)], C[bn], SG[bn]).wait()

    def launch_w(k, b):
        base = wid * epw + k * CB
        pltpu.async_copy(M[b], m_out.at[pl.ds(base, CB)], SW[b])

    def wait_w(k, b):
        base = wid * epw + k * CB
        pltpu.make_async_copy(M[b], m_out.at[pl.ds(base, CB)], SW[b]).wait()

    def compute(k, b):
        av, bv, cv, mv = A[b], B[b], C[b], M[b]

        def edge_body(e, c2):
            for cg in range(16):
                sl = pl.ds(cg * 16, 16)
                m = av[e, sl] + bv[e, sl] + cv[e, sl]
                mv[e, sl] = m
                acc_v[cg] = acc_v[cg] + m
                acc_v[16 + cg] = acc_v[16 + cg] + m * m
            return c2

        lax.fori_loop(0, CB, edge_body, 0, unroll=False)

    base0 = wid * epw
    pltpu.sync_copy(src.at[pl.ds(base0, CB)], SI[0])
    pltpu.sync_copy(dst.at[pl.ds(base0, CB)], DI[0])
    launch_g(0, 0)
    launch_i(1, 1)
    ng = nchunk // 2

    def body(g, carry):
        for j in range(2):
            k = 2 * g + j
            b = j
            bn = 1 - j

            def adv():
                wait_i(k + 1, bn)
                launch_g(k + 1, bn)

            if j == 0:
                adv()
            else:
                pl.when(g < ng - 1)(adv)

            @pl.when(g > 0)
            def _():
                wait_w(k - 2, b)

            wait_g(k, b)

            @pl.when(g < ng - 1)
            def _():
                launch_i(k + 2, b)

            compute(k, b)
            launch_w(k, b)
        return carry

    lax.fori_loop(0, ng, body, 0, unroll=False)
    wait_w(nchunk - 2, 0)
    wait_w(nchunk - 1, 1)
    pltpu.sync_copy(acc_v, stats_out.at[wid])


def _s3_body(n_pad, epw, nchunk, m_in, dst, params, out,
             di0, di1, di2, di3, di4, mm0, mm1, mm2, mm3, mm4,
             ac0, ac1, ac2, ac3, ac4,
             par_v, h_sh,
             sr0, sr1, sr2, sr3, sr4, ss0, ss1, ss2, ss3, ss4):
    cid = lax.axis_index("c")
    sid = lax.axis_index("s")
    wid = sid * NC + cid
    rows_per_tile = n_pad // NS
    DI = (di0, di1, di2, di3, di4)
    M = (mm0, mm1, mm2, mm3, mm4)
    ACT = (ac0, ac1, ac2, ac3, ac4)
    SR = (sr0, sr1, sr2, sr3, sr4)
    SS = (ss0, ss1, ss2, ss3, ss4)

    pltpu.sync_copy(params, par_v)

    def zrow(r, c2):
        for g in range(8):
            ac0[r, pl.ds(g * 16, 16)] = jnp.zeros((16,), jnp.float32)
        return c2

    lax.fori_loop(0, CB3, zrow, 0, unroll=False)

    def zcopy(i, c2):
        pltpu.sync_copy(ac0, h_sh.at[pl.ds(sid * rows_per_tile + i * CB3, CB3)])
        return c2

    lax.fori_loop(0, rows_per_tile // CB3, zcopy, 0, unroll=False)
    plsc.subcore_barrier()

    def launch_r(k, b2, b5):
        base = wid * epw + k * CB3
        pltpu.async_copy(m_in.at[pl.ds(base, CB3)], M[b2], SR[b2])
        pltpu.async_copy(dst.at[pl.ds(base, CB3)], DI[b5], SR[b2])

    def wait_r(k, b2, b5):
        base = wid * epw + k * CB3
        pltpu.make_async_copy(m_in.at[pl.ds(base, CB3)], M[b2], SR[b2]).wait()
        pltpu.make_async_copy(dst.at[pl.ds(base, CB3)], DI[b5], SR[b2]).wait()

    def launch_s(k, b5):
        pltpu.async_copy(ACT[b5], h_sh.at[DI[b5]], SS[b5], add=True)

    def wait_s(k, b5):
        pltpu.make_async_copy(ACT[b5], h_sh.at[DI[b5]], SS[b5]).wait()

    def compute(k, b2, b5):
        mv, av = M[b2], ACT[b5]

        def edge_body(e, c2):
            for cg in range(8):
                slf = pl.ds(cg * 16, 16)
                sls = pl.ds(128 + cg * 16, 16)
                f = mv[e, slf] * par_v[0, slf] + par_v[1, slf]
                s = mv[e, sls] * par_v[0, sls] + par_v[1, sls]
                sig = 1.0 / (1.0 + jnp.exp(-f))
                t = jnp.exp(-jnp.abs(s))
                sp = jnp.maximum(s, 0.0) + _log1p_estrin(t)
                av[e, slf] = sig * sp
            return c2

        lax.fori_loop(0, CB3, edge_body, 0, unroll=False)

    launch_r(0, 0, 0)
    ng = nchunk // 5

    def body(g, carry):
        for j in range(5):
            k = 5 * g + j
            b5 = j
            bn5 = (j + 1) % 5

            def drain():
                wait_s(k - 4, bn5)

            if j >= 4:
                drain()
            else:
                pl.when(g > 0)(drain)

            def adv():
                launch_r(k + 1, bn5, bn5)

            if j == 4:
                pl.when(g < ng - 1)(adv)
            else:
                adv()

            wait_r(k, b5, b5)
            compute(k, b5, b5)
            launch_s(k, b5)
        return carry

    lax.fori_loop(0, ng, body, 0, unroll=False)
    for k in range(nchunk - 4, nchunk):
        wait_s(k, k % 5)
    plsc.subcore_barrier()
    r0 = sid * rows_per_tile
    pltpu.sync_copy(h_sh.at[pl.ds(r0, rows_per_tile)],
                    out.at[cid, pl.ds(r0, rows_per_tile)])


def _g4_body(p_ref, node_ref, gn_ref, bn_ref, out_ref):
    h = p_ref[0] + p_ref[1]
    mean = jnp.mean(h, axis=0, keepdims=True)
    var = jnp.mean((h - mean) ** 2, axis=0, keepdims=True)
    hn = gn_ref[...] * (h - mean) * lax.rsqrt(var + 1e-5) + bn_ref[...]
    x = node_ref[...] + hn
    out_ref[...] = jnp.maximum(x, 0.0) + jnp.log1p(jnp.exp(-jnp.abs(x)))


def kernel(node_feats, edge_feats, edge_index, W_src, b_src, W_dst, b_dst,
           W_edge, b_edge, gamma_m, beta_m, gamma_n, beta_n):
    n_nodes, d = node_feats.shape
    n_edges, de = edge_feats.shape
    d2 = 2 * d
    assert n_edges % NW == 0
    epw = n_edges // NW
    assert epw % CB == 0
    nchunk = epw // CB
    assert nchunk % 2 == 0 and nchunk >= 4
    assert epw % CB3 == 0
    nchunk3 = epw // CB3
    assert nchunk3 % 5 == 0 and nchunk3 >= 10
    n_pad = ((n_nodes + NS * 40 - 1) // (NS * 40)) * (NS * 40)
    assert (n_pad // NS) % CB3 == 0

    src = edge_index[0]
    dst = edge_index[1]

    nb = 10
    bn_rows = n_nodes // nb
    h_src, h_dst = pl.pallas_call(
        _proj_body,
        grid=(nb,),
        in_specs=[
            pl.BlockSpec((bn_rows, d), lambda i: (i, 0)),
            pl.BlockSpec((d, d2), lambda i: (0, 0)),
            pl.BlockSpec((1, d2), lambda i: (0, 0)),
            pl.BlockSpec((d, d2), lambda i: (0, 0)),
            pl.BlockSpec((1, d2), lambda i: (0, 0)),
        ],
        out_specs=[
            pl.BlockSpec((bn_rows, d2), lambda i: (i, 0)),
            pl.BlockSpec((bn_rows, d2), lambda i: (i, 0)),
        ],
        out_shape=[
            jax.ShapeDtypeStruct((n_nodes, d2), jnp.float32),
            jax.ShapeDtypeStruct((n_nodes, d2), jnp.float32),
        ],
    )(node_feats, W_src, b_src.reshape(1, d2), W_dst, b_dst.reshape(1, d2))

    eb = 80
    be_rows = n_edges // eb
    eproj = pl.pallas_call(
        _eproj_body,
        grid=(eb,),
        in_specs=[
            pl.BlockSpec((be_rows, de), lambda i: (i, 0)),
            pl.BlockSpec((de, d2), lambda i: (0, 0)),
            pl.BlockSpec((1, d2), lambda i: (0, 0)),
        ],
        out_specs=pl.BlockSpec((be_rows, d2), lambda i: (i, 0)),
        out_shape=jax.ShapeDtypeStruct((n_edges, d2), jnp.float32),
    )(edge_feats, W_edge, b_edge.reshape(1, d2))

    mesh = plsc.VectorSubcoreMesh(core_axis_name="c", subcore_axis_name="s")
    s2 = functools.partial(
        pl.kernel,
        out_type=(
            jax.ShapeDtypeStruct((n_edges, d2), jnp.float32),
            jax.ShapeDtypeStruct((NW, 32, 16), jnp.float32),
        ),
        mesh=mesh,
        scratch_types=(
            [pltpu.VMEM((CB,), jnp.int32)] * 4
            + [pltpu.VMEM((CB, d2), jnp.float32)] * 8
            + [pltpu.VMEM((32, 16), jnp.float32)]
            + [pltpu.SemaphoreType.DMA] * 6
        ),
    )(functools.partial(_s2_body, epw, nchunk))
    m_arr, stats = s2(h_src, h_dst, eproj, src, dst)

    ssum = stats.sum(axis=0)
    sum_m = ssum[:16].reshape(d2)
    sum_sq = ssum[16:].reshape(d2)
    mean = sum_m / n_edges
    var = jnp.maximum(sum_sq / n_edges - mean * mean, 0.0)
    scale = gamma_m * lax.rsqrt(var + 1e-5)
    shift = beta_m - mean * scale
    params = jnp.stack([scale, shift])

    s3 = functools.partial(
        pl.kernel,
        out_type=jax.ShapeDtypeStruct((NC, n_pad, d), jnp.float32),
        mesh=mesh,
        scratch_types=(
            [pltpu.VMEM((CB3,), jnp.int32)] * 5
            + [pltpu.VMEM((CB3, d2), jnp.float32)] * 5
            + [pltpu.VMEM((CB3, d), jnp.float32)] * 5
            + [pltpu.VMEM((2, d2), jnp.float32)]
            + [pltpu.VMEM_SHARED((n_pad, d), jnp.float32)]
            + [pltpu.SemaphoreType.DMA] * 10
        ),
    )(functools.partial(_s3_body, n_pad, epw, nchunk3))
    partials = s3(m_arr, dst, params)

    out = pl.pallas_call(
        _g4_body,
        grid=(1,),
        in_specs=[
            pl.BlockSpec((NC, n_nodes, d), lambda i: (0, 0, 0)),
            pl.BlockSpec((n_nodes, d), lambda i: (0, 0)),
            pl.BlockSpec((1, d), lambda i: (0, 0)),
            pl.BlockSpec((1, d), lambda i: (0, 0)),
        ],
        out_specs=pl.BlockSpec((n_nodes, d), lambda i: (0, 0)),
        out_shape=jax.ShapeDtypeStruct((n_nodes, d), jnp.float32),
    )(partials, node_feats, gamma_n.reshape(1, d), beta_n.reshape(1, d))
    return out


# speedup vs baseline: 1.9125x; 1.9125x over previous
import functools

import jax
import jax.numpy as jnp
from jax import lax
from jax.experimental import pallas as pl
from jax.experimental.pallas import tpu as pltpu
from jax.experimental.pallas import tpu_sc as plsc

NC = 2
NS = 16
NW = NC * NS
CB = 40
CB3 = 40

_LOG1P_C = (
    3.3869654e-08, 9.9999428e-01, -4.9983856e-01, 3.3154863e-01,
    -2.3982616e-01, 1.6582276e-01, -9.3252040e-02, 3.4849711e-02,
    -6.1514708e-03,
)


def _log1p_poly(t):
    y = jnp.full_like(t, _LOG1P_C[-1])
    for c in _LOG1P_C[-2::-1]:
        y = y * t + c
    return y


def _log1p_estrin(t):
    c = _LOG1P_C
    t2 = t * t
    t4 = t2 * t2
    p01 = c[0] + c[1] * t
    p23 = c[2] + c[3] * t
    p45 = c[4] + c[5] * t
    p67 = c[6] + c[7] * t
    q0 = p01 + p23 * t2
    q1 = p45 + p67 * t2
    return q0 + (q1 + c[8] * t4) * t4


def _proj_body(node_ref, ws_ref, bs_ref, wd_ref, bd_ref, hs_ref, hd_ref):
    x = node_ref[...]
    hs_ref[...] = jnp.dot(x, ws_ref[...], preferred_element_type=jnp.float32) + bs_ref[...]
    hd_ref[...] = jnp.dot(x, wd_ref[...], preferred_element_type=jnp.float32) + bd_ref[...]


def _eproj_body(ef_ref, we_ref, be_ref, out_ref):
    out_ref[...] = (
        jnp.dot(ef_ref[...], we_ref[...], preferred_element_type=jnp.float32)
        + be_ref[...]
    )


def _s2_body(epw, nchunk, hsrc, hdst, eproj, src, dst,
             m_out, stats_out,
             si0, si1, di0, di1, a0, a1, b0, b1, c0, c1, m0, m1, acc_v,
             semg0, semg1, semw0, semw1, semi0, semi1):
    cid = lax.axis_index("c")
    sid = lax.axis_index("s")
    wid = sid * NC + cid
    SI = (si0, si1)
    DI = (di0, di1)
    A = (a0, a1)
    B = (b0, b1)
    C = (c0, c1)
    M = (m0, m1)
    SG = (semg0, semg1)
    SW = (semw0, semw1)
    SEMI = (semi0, semi1)

    for r in range(32):
        acc_v[r] = jnp.zeros((16,), jnp.float32)

    def launch_i(k, b):
        base = wid * epw + k * CB
        pltpu.async_copy(src.at[pl.ds(base, CB)], SI[b], SEMI[b])
        pltpu.async_copy(dst.at[pl.ds(base, CB)], DI[b], SEMI[b])

    def wait_i(k, b):
        base = wid * epw + k * CB
        pltpu.make_async_copy(src.at[pl.ds(base, CB)], SI[b], SEMI[b]).wait()
        pltpu.make_async_copy(dst.at[pl.ds(base, CB)], DI[b], SEMI[b]).wait()

    def launch_g(k, bn):
        base = wid * epw + k * CB
        pltpu.async_copy(hsrc.at[SI[bn]], A[bn], SG[bn])
        pltpu.async_copy(hdst.at[DI[bn]], B[bn], SG[bn])
        pltpu.async_copy(eproj.at[pl.ds(base, CB)], C[bn], SG[bn])

    def wait_g(k, bn):
        base = wid * epw + k * CB
        pltpu.make_async_copy(hsrc.at[SI[bn]], A[bn], SG[bn]).wait()
        pltpu.make_async_copy(hdst.at[DI[bn]], B[bn], SG[bn]).wait()
        pltpu.make_async_copy(eproj.at[pl.ds(base, CB)], C[bn], SG[bn]).wait()

    def launch_w(k, b):
        base = wid * epw + k * CB
        pltpu.async_copy(M[b], m_out.at[pl.ds(base, CB)], SW[b])

    def wait_w(k, b):
        base = wid * epw + k * CB
        pltpu.make_async_copy(M[b], m_out.at[pl.ds(base, CB)], SW[b]).wait()

    def compute(k, b):
        av, bv, cv, mv = A[b], B[b], C[b], M[b]

        def edge_body(e, c2):
            for cg in range(16):
                sl = pl.ds(cg * 16, 16)
                m = av[e, sl] + bv[e, sl] + cv[e, sl]
                mv[e, sl] = m
                acc_v[cg] = acc_v[cg] + m
                acc_v[16 + cg] = acc_v[16 + cg] + m * m
            return c2

        lax.fori_loop(0, CB, edge_body, 0, unroll=False)

    base0 = wid * epw
    pltpu.sync_copy(src.at[pl.ds(base0, CB)], SI[0])
    pltpu.sync_copy(dst.at[pl.ds(base0, CB)], DI[0])
    launch_g(0, 0)
    launch_i(1, 1)
    ng = nchunk // 2

    def body(g, carry):
        for j in range(2):
            k = 2 * g + j
            b = j
            bn = 1 - j

            def adv():
                wait_i(k + 1, bn)
                launch_g(k + 1, bn)

            if j == 0:
                adv()
            else:
                pl.when(g < ng - 1)(adv)

            @pl.when(g > 0)
            def _():
                wait_w(k - 2, b)

            wait_g(k, b)

            @pl.when(g < ng - 1)
            def _():
                launch_i(k + 2, b)

            compute(k, b)
            launch_w(k, b)
        return carry

    lax.fori_loop(0, ng, body, 0, unroll=False)
    wait_w(nchunk - 2, 0)
    wait_w(nchunk - 1, 1)
    pltpu.sync_copy(acc_v, stats_out.at[wid])


def _act_body(m_ref, par_ref, out_ref):
    z = m_ref[...] * par_ref[0:1, :] + par_ref[1:2, :]
    f = z[:, :128]
    s = z[:, 128:]
    sig = 1.0 / (1.0 + jnp.exp(-f))
    sp = jnp.maximum(s, 0.0) + jnp.log1p(jnp.exp(-jnp.abs(s)))
    out_ref[...] = sig * sp


def _s3_body(n_pad, epw, nchunk, act_in, dst, out,
             di0, di1, di2, di3, di4, ac0, ac1, ac2, ac3, ac4,
             h_sh,
             sr0, sr1, sr2, sr3, sr4, ss0, ss1, ss2, ss3, ss4):
    cid = lax.axis_index("c")
    sid = lax.axis_index("s")
    wid = sid * NC + cid
    rows_per_tile = n_pad // NS
    DI = (di0, di1, di2, di3, di4)
    ACT = (ac0, ac1, ac2, ac3, ac4)
    SR = (sr0, sr1, sr2, sr3, sr4)
    SS = (ss0, ss1, ss2, ss3, ss4)

    def zrow(r, c2):
        for g in range(8):
            ac0[r, pl.ds(g * 16, 16)] = jnp.zeros((16,), jnp.float32)
        return c2

    lax.fori_loop(0, CB3, zrow, 0, unroll=False)

    def zcopy(i, c2):
        pltpu.sync_copy(ac0, h_sh.at[pl.ds(sid * rows_per_tile + i * CB3, CB3)])
        return c2

    lax.fori_loop(0, rows_per_tile // CB3, zcopy, 0, unroll=False)
    plsc.subcore_barrier()

    def launch_r(k, b):
        base = wid * epw + k * CB3
        pltpu.async_copy(act_in.at[pl.ds(base, CB3)], ACT[b], SR[b])
        pltpu.async_copy(dst.at[pl.ds(base, CB3)], DI[b], SR[b])

    def wait_r(k, b):
        base = wid * epw + k * CB3
        pltpu.make_async_copy(act_in.at[pl.ds(base, CB3)], ACT[b], SR[b]).wait()
        pltpu.make_async_copy(dst.at[pl.ds(base, CB3)], DI[b], SR[b]).wait()

    def launch_s(k, b):
        pltpu.async_copy(ACT[b], h_sh.at[DI[b]], SS[b], add=True)

    def wait_s(k, b):
        pltpu.make_async_copy(ACT[b], h_sh.at[DI[b]], SS[b]).wait()

    launch_r(0, 0)
    ng = nchunk // 5

    def body(g, carry):
        for j in range(5):
            k = 5 * g + j
            b = j
            bn = (j + 1) % 5

            def drain():
                wait_s(k - 4, bn)

            if j >= 4:
                drain()
            else:
                pl.when(g > 0)(drain)

            def adv():
                launch_r(k + 1, bn)

            if j == 4:
                pl.when(g < ng - 1)(adv)
            else:
                adv()

            wait_r(k, b)
            launch_s(k, b)
        return carry

    lax.fori_loop(0, ng, body, 0, unroll=False)
    for k in range(nchunk - 4, nchunk):
        wait_s(k, k % 5)
    plsc.subcore_barrier()
    r0 = sid * rows_per_tile
    pltpu.sync_copy(h_sh.at[pl.ds(r0, rows_per_tile)],
                    out.at[cid, pl.ds(r0, rows_per_tile)])


def _g4_body(p_ref, node_ref, gn_ref, bn_ref, out_ref):
    h = p_ref[0] + p_ref[1]
    mean = jnp.mean(h, axis=0, keepdims=True)
    var = jnp.mean((h - mean) ** 2, axis=0, keepdims=True)
    hn = gn_ref[...] * (h - mean) * lax.rsqrt(var + 1e-5) + bn_ref[...]
    x = node_ref[...] + hn
    out_ref[...] = jnp.maximum(x, 0.0) + jnp.log1p(jnp.exp(-jnp.abs(x)))


def kernel(node_feats, edge_feats, edge_index, W_src, b_src, W_dst, b_dst,
           W_edge, b_edge, gamma_m, beta_m, gamma_n, beta_n):
    n_nodes, d = node_feats.shape
    n_edges, de = edge_feats.shape
    d2 = 2 * d
    assert n_edges % NW == 0
    epw = n_edges // NW
    assert epw % CB == 0
    nchunk = epw // CB
    assert nchunk % 2 == 0 and nchunk >= 4
    assert epw % CB3 == 0
    nchunk3 = epw // CB3
    assert nchunk3 % 5 == 0 and nchunk3 >= 10
    n_pad = ((n_nodes + NS * 40 - 1) // (NS * 40)) * (NS * 40)
    assert (n_pad // NS) % CB3 == 0

    src = edge_index[0]
    dst = edge_index[1]

    nb = 10
    bn_rows = n_nodes // nb
    h_src, h_dst = pl.pallas_call(
        _proj_body,
        grid=(nb,),
        in_specs=[
            pl.BlockSpec((bn_rows, d), lambda i: (i, 0)),
            pl.BlockSpec((d, d2), lambda i: (0, 0)),
            pl.BlockSpec((1, d2), lambda i: (0, 0)),
            pl.BlockSpec((d, d2), lambda i: (0, 0)),
            pl.BlockSpec((1, d2), lambda i: (0, 0)),
        ],
        out_specs=[
            pl.BlockSpec((bn_rows, d2), lambda i: (i, 0)),
            pl.BlockSpec((bn_rows, d2), lambda i: (i, 0)),
        ],
        out_shape=[
            jax.ShapeDtypeStruct((n_nodes, d2), jnp.float32),
            jax.ShapeDtypeStruct((n_nodes, d2), jnp.float32),
        ],
    )(node_feats, W_src, b_src.reshape(1, d2), W_dst, b_dst.reshape(1, d2))

    eb = 80
    be_rows = n_edges // eb
    eproj = pl.pallas_call(
        _eproj_body,
        grid=(eb,),
        in_specs=[
            pl.BlockSpec((be_rows, de), lambda i: (i, 0)),
            pl.BlockSpec((de, d2), lambda i: (0, 0)),
            pl.BlockSpec((1, d2), lambda i: (0, 0)),
        ],
        out_specs=pl.BlockSpec((be_rows, d2), lambda i: (i, 0)),
        out_shape=jax.ShapeDtypeStruct((n_edges, d2), jnp.float32),
    )(edge_feats, W_edge, b_edge.reshape(1, d2))

    mesh = plsc.VectorSubcoreMesh(core_axis_name="c", subcore_axis_name="s")
    s2 = functools.partial(
        pl.kernel,
        out_type=(
            jax.ShapeDtypeStruct((n_edges, d2), jnp.float32),
            jax.ShapeDtypeStruct((NW, 32, 16), jnp.float32),
        ),
        mesh=mesh,
        scratch_types=(
            [pltpu.VMEM((CB,), jnp.int32)] * 4
            + [pltpu.VMEM((CB, d2), jnp.float32)] * 8
            + [pltpu.VMEM((32, 16), jnp.float32)]
            + [pltpu.SemaphoreType.DMA] * 6
        ),
    )(functools.partial(_s2_body, epw, nchunk))
    m_arr, stats = s2(h_src, h_dst, eproj, src, dst)

    ssum = stats.sum(axis=0)
    sum_m = ssum[:16].reshape(d2)
    sum_sq = ssum[16:].reshape(d2)
    mean = sum_m / n_edges
    var = jnp.maximum(sum_sq / n_edges - mean * mean, 0.0)
    scale = gamma_m * lax.rsqrt(var + 1e-5)
    shift = beta_m - mean * scale
    params = jnp.stack([scale, shift])

    ab = 80
    ab_rows = n_edges // ab
    act = pl.pallas_call(
        _act_body,
        grid=(ab,),
        in_specs=[
            pl.BlockSpec((ab_rows, d2), lambda i: (i, 0)),
            pl.BlockSpec((2, d2), lambda i: (0, 0)),
        ],
        out_specs=pl.BlockSpec((ab_rows, d), lambda i: (i, 0)),
        out_shape=jax.ShapeDtypeStruct((n_edges, d), jnp.float32),
    )(m_arr, params)

    s3 = functools.partial(
        pl.kernel,
        out_type=jax.ShapeDtypeStruct((NC, n_pad, d), jnp.float32),
        mesh=mesh,
        scratch_types=(
            [pltpu.VMEM((CB3,), jnp.int32)] * 5
            + [pltpu.VMEM((CB3, d), jnp.float32)] * 5
            + [pltpu.VMEM_SHARED((n_pad, d), jnp.float32)]
            + [pltpu.SemaphoreType.DMA] * 10
        ),
    )(functools.partial(_s3_body, n_pad, epw, nchunk3))
    partials = s3(act, dst)

    out = pl.pallas_call(
        _g4_body,
        grid=(1,),
        in_specs=[
            pl.BlockSpec((NC, n_nodes, d), lambda i: (0, 0, 0)),
            pl.BlockSpec((n_nodes, d), lambda i: (0, 0)),
            pl.BlockSpec((1, d), lambda i: (0, 0)),
            pl.BlockSpec((1, d), lambda i: (0, 0)),
        ],
        out_specs=pl.BlockSpec((n_nodes, d), lambda i: (0, 0)),
        out_shape=jax.ShapeDtypeStruct((n_nodes, d), jnp.float32),
    )(partials, node_feats, gamma_n.reshape(1, d), beta_n.reshape(1, d))
    return out


# speedup vs baseline: 4.7326x; 2.4746x over previous
import functools

import jax
import jax.numpy as jnp
from jax import lax
from jax.experimental import pallas as pl
from jax.experimental.pallas import tpu as pltpu
from jax.experimental.pallas import tpu_sc as plsc

NC = 2
NS = 16
NW = NC * NS
CB = 40
CB3 = 40

_LOG1P_C = (
    3.3869654e-08, 9.9999428e-01, -4.9983856e-01, 3.3154863e-01,
    -2.3982616e-01, 1.6582276e-01, -9.3252040e-02, 3.4849711e-02,
    -6.1514708e-03,
)


def _log1p_poly(t):
    y = jnp.full_like(t, _LOG1P_C[-1])
    for c in _LOG1P_C[-2::-1]:
        y = y * t + c
    return y


def _log1p_estrin(t):
    c = _LOG1P_C
    t2 = t * t
    t4 = t2 * t2
    p01 = c[0] + c[1] * t
    p23 = c[2] + c[3] * t
    p45 = c[4] + c[5] * t
    p67 = c[6] + c[7] * t
    q0 = p01 + p23 * t2
    q1 = p45 + p67 * t2
    return q0 + (q1 + c[8] * t4) * t4


def _proj_body(node_ref, ws_ref, bs_ref, wd_ref, bd_ref, hs_ref, hd_ref):
    x = node_ref[...]
    hs_ref[...] = jnp.dot(x, ws_ref[...], preferred_element_type=jnp.float32) + bs_ref[...]
    hd_ref[...] = jnp.dot(x, wd_ref[...], preferred_element_type=jnp.float32) + bd_ref[...]


def _s2_body(epw, nchunk, hsrc, hdst, src, dst, m_out,
             si0, si1, di0, di1, a0, a1, b0, b1, m0, m1,
             semg0, semg1, semw0, semw1, semi0, semi1):
    cid = lax.axis_index("c")
    sid = lax.axis_index("s")
    wid = sid * NC + cid
    SI = (si0, si1)
    DI = (di0, di1)
    A = (a0, a1)
    B = (b0, b1)
    M = (m0, m1)
    SG = (semg0, semg1)
    SW = (semw0, semw1)
    SEMI = (semi0, semi1)

    def launch_i(k, b):
        base = wid * epw + k * CB
        pltpu.async_copy(src.at[pl.ds(base, CB)], SI[b], SEMI[b])
        pltpu.async_copy(dst.at[pl.ds(base, CB)], DI[b], SEMI[b])

    def wait_i(k, b):
        base = wid * epw + k * CB
        pltpu.make_async_copy(src.at[pl.ds(base, CB)], SI[b], SEMI[b]).wait()
        pltpu.make_async_copy(dst.at[pl.ds(base, CB)], DI[b], SEMI[b]).wait()

    def launch_g(k, bn):
        pltpu.async_copy(hsrc.at[SI[bn]], A[bn], SG[bn])
        pltpu.async_copy(hdst.at[DI[bn]], B[bn], SG[bn])

    def wait_g(k, bn):
        pltpu.make_async_copy(hsrc.at[SI[bn]], A[bn], SG[bn]).wait()
        pltpu.make_async_copy(hdst.at[DI[bn]], B[bn], SG[bn]).wait()

    def launch_w(k, b):
        base = wid * epw + k * CB
        pltpu.async_copy(M[b], m_out.at[pl.ds(base, CB)], SW[b])

    def wait_w(k, b):
        base = wid * epw + k * CB
        pltpu.make_async_copy(M[b], m_out.at[pl.ds(base, CB)], SW[b]).wait()

    def compute(k, b):
        av, bv, mv = A[b], B[b], M[b]

        def edge_body(e, c2):
            for cg in range(16):
                sl = pl.ds(cg * 16, 16)
                mv[e, sl] = av[e, sl] + bv[e, sl]
            return c2

        lax.fori_loop(0, CB, edge_body, 0, unroll=False)

    base0 = wid * epw
    pltpu.sync_copy(src.at[pl.ds(base0, CB)], SI[0])
    pltpu.sync_copy(dst.at[pl.ds(base0, CB)], DI[0])
    launch_g(0, 0)
    launch_i(1, 1)
    ng = nchunk // 2

    def body(g, carry):
        for j in range(2):
            k = 2 * g + j
            b = j
            bn = 1 - j

            def adv():
                wait_i(k + 1, bn)
                launch_g(k + 1, bn)

            if j == 0:
                adv()
            else:
                pl.when(g < ng - 1)(adv)

            @pl.when(g > 0)
            def _():
                wait_w(k - 2, b)

            wait_g(k, b)

            @pl.when(g < ng - 1)
            def _():
                launch_i(k + 2, b)

            compute(k, b)
            launch_w(k, b)
        return carry

    lax.fori_loop(0, ng, body, 0, unroll=False)
    wait_w(nchunk - 2, 0)
    wait_w(nchunk - 1, 1)


def _stats_body(mp_ref, ef_ref, we_ref, be_ref, out_ref):
    i = pl.program_id(0)
    c = jnp.dot(ef_ref[...], we_ref[...], preferred_element_type=jnp.float32)
    m = mp_ref[...] + c + be_ref[...]
    s1 = jnp.sum(m, axis=0, keepdims=True)
    s2 = jnp.sum(m * m, axis=0, keepdims=True)
    blk = jnp.concatenate([s1, s2], axis=0)

    @pl.when(i == 0)
    def _():
        out_ref[...] = blk

    @pl.when(i > 0)
    def _():
        out_ref[...] = out_ref[...] + blk


def _act_body(mp_ref, ef_ref, we_ref, be_ref, par_ref, out_ref):
    c = jnp.dot(ef_ref[...], we_ref[...], preferred_element_type=jnp.float32)
    m = mp_ref[...] + c + be_ref[...]
    z = m * par_ref[0:1, :] + par_ref[1:2, :]
    f = z[:, :128]
    s = z[:, 128:]
    sig = 1.0 / (1.0 + jnp.exp(-f))
    sp = jnp.maximum(s, 0.0) + jnp.log1p(jnp.exp(-jnp.abs(s)))
    out_ref[...] = sig * sp


def _s3_body(n_pad, epw, nchunk, act_in, dst, out,
             di0, di1, di2, di3, di4, ac0, ac1, ac2, ac3, ac4,
             h_sh,
             sr0, sr1, sr2, sr3, sr4, ss0, ss1, ss2, ss3, ss4):
    cid = lax.axis_index("c")
    sid = lax.axis_index("s")
    wid = sid * NC + cid
    rows_per_tile = n_pad // NS
    DI = (di0, di1, di2, di3, di4)
    ACT = (ac0, ac1, ac2, ac3, ac4)
    SR = (sr0, sr1, sr2, sr3, sr4)
    SS = (ss0, ss1, ss2, ss3, ss4)

    def zrow(r, c2):
        for g in range(8):
            ac0[r, pl.ds(g * 16, 16)] = jnp.zeros((16,), jnp.float32)
        return c2

    lax.fori_loop(0, CB3, zrow, 0, unroll=False)

    def zcopy(i, c2):
        pltpu.sync_copy(ac0, h_sh.at[pl.ds(sid * rows_per_tile + i * CB3, CB3)])
        return c2

    lax.fori_loop(0, rows_per_tile // CB3, zcopy, 0, unroll=False)
    plsc.subcore_barrier()

    def launch_r(k, b):
        base = wid * epw + k * CB3
        pltpu.async_copy(act_in.at[pl.ds(base, CB3)], ACT[b], SR[b])
        pltpu.async_copy(dst.at[pl.ds(base, CB3)], DI[b], SR[b])

    def wait_r(k, b):
        base = wid * epw + k * CB3
        pltpu.make_async_copy(act_in.at[pl.ds(base, CB3)], ACT[b], SR[b]).wait()
        pltpu.make_async_copy(dst.at[pl.ds(base, CB3)], DI[b], SR[b]).wait()

    def launch_s(k, b):
        pltpu.async_copy(ACT[b], h_sh.at[DI[b]], SS[b], add=True)

    def wait_s(k, b):
        pltpu.make_async_copy(ACT[b], h_sh.at[DI[b]], SS[b]).wait()

    launch_r(0, 0)
    ng = nchunk // 5

    def body(g, carry):
        for j in range(5):
            k = 5 * g + j
            b = j
            bn = (j + 1) % 5

            def drain():
                wait_s(k - 4, bn)

            if j >= 4:
                drain()
            else:
                pl.when(g > 0)(drain)

            def adv():
                launch_r(k + 1, bn)

            if j == 4:
                pl.when(g < ng - 1)(adv)
            else:
                adv()

            wait_r(k, b)
            launch_s(k, b)
        return carry

    lax.fori_loop(0, ng, body, 0, unroll=False)
    for k in range(nchunk - 4, nchunk):
        wait_s(k, k % 5)
    plsc.subcore_barrier()
    r0 = sid * rows_per_tile
    pltpu.sync_copy(h_sh.at[pl.ds(r0, rows_per_tile)],
                    out.at[cid, pl.ds(r0, rows_per_tile)])


def _g4_body(p_ref, node_ref, gn_ref, bn_ref, out_ref):
    h = p_ref[0] + p_ref[1]
    mean = jnp.mean(h, axis=0, keepdims=True)
    var = jnp.mean((h - mean) ** 2, axis=0, keepdims=True)
    hn = gn_ref[...] * (h - mean) * lax.rsqrt(var + 1e-5) + bn_ref[...]
    x = node_ref[...] + hn
    out_ref[...] = jnp.maximum(x, 0.0) + jnp.log1p(jnp.exp(-jnp.abs(x)))


def kernel(node_feats, edge_feats, edge_index, W_src, b_src, W_dst, b_dst,
           W_edge, b_edge, gamma_m, beta_m, gamma_n, beta_n):
    n_nodes, d = node_feats.shape
    n_edges, de = edge_feats.shape
    d2 = 2 * d
    assert n_edges % NW == 0
    epw = n_edges // NW
    assert epw % CB == 0
    nchunk = epw // CB
    assert nchunk % 2 == 0 and nchunk >= 4
    assert epw % CB3 == 0
    nchunk3 = epw // CB3
    assert nchunk3 % 5 == 0 and nchunk3 >= 10
    n_pad = ((n_nodes + NS * 40 - 1) // (NS * 40)) * (NS * 40)
    assert (n_pad // NS) % CB3 == 0

    src = edge_index[0]
    dst = edge_index[1]

    nb = 10
    bn_rows = n_nodes // nb
    h_src, h_dst = pl.pallas_call(
        _proj_body,
        grid=(nb,),
        in_specs=[
            pl.BlockSpec((bn_rows, d), lambda i: (i, 0)),
            pl.BlockSpec((d, d2), lambda i: (0, 0)),
            pl.BlockSpec((1, d2), lambda i: (0, 0)),
            pl.BlockSpec((d, d2), lambda i: (0, 0)),
            pl.BlockSpec((1, d2), lambda i: (0, 0)),
        ],
        out_specs=[
            pl.BlockSpec((bn_rows, d2), lambda i: (i, 0)),
            pl.BlockSpec((bn_rows, d2), lambda i: (i, 0)),
        ],
        out_shape=[
            jax.ShapeDtypeStruct((n_nodes, d2), jnp.float32),
            jax.ShapeDtypeStruct((n_nodes, d2), jnp.float32),
        ],
    )(node_feats, W_src, b_src.reshape(1, d2), W_dst, b_dst.reshape(1, d2))

    mesh = plsc.VectorSubcoreMesh(core_axis_name="c", subcore_axis_name="s")
    s2 = functools.partial(
        pl.kernel,
        out_type=jax.ShapeDtypeStruct((n_edges, d2), jnp.float32),
        mesh=mesh,
        scratch_types=(
            [pltpu.VMEM((CB,), jnp.int32)] * 4
            + [pltpu.VMEM((CB, d2), jnp.float32)] * 6
            + [pltpu.SemaphoreType.DMA] * 6
        ),
    )(functools.partial(_s2_body, epw, nchunk))
    m_arr = s2(h_src, h_dst, src, dst)

    eb = 80
    be_rows = n_edges // eb
    ssum = pl.pallas_call(
        _stats_body,
        grid=(eb,),
        in_specs=[
            pl.BlockSpec((be_rows, d2), lambda i: (i, 0)),
            pl.BlockSpec((be_rows, de), lambda i: (i, 0)),
            pl.BlockSpec((de, d2), lambda i: (0, 0)),
            pl.BlockSpec((1, d2), lambda i: (0, 0)),
        ],
        out_specs=pl.BlockSpec((2, d2), lambda i: (0, 0)),
        out_shape=jax.ShapeDtypeStruct((2, d2), jnp.float32),
    )(m_arr, edge_feats, W_edge, b_edge.reshape(1, d2))

    mean = ssum[0] / n_edges
    var = jnp.maximum(ssum[1] / n_edges - mean * mean, 0.0)
    scale = gamma_m * lax.rsqrt(var + 1e-5)
    shift = beta_m - mean * scale
    params = jnp.stack([scale, shift])

    act = pl.pallas_call(
        _act_body,
        grid=(eb,),
        in_specs=[
            pl.BlockSpec((be_rows, d2), lambda i: (i, 0)),
            pl.BlockSpec((be_rows, de), lambda i: (i, 0)),
            pl.BlockSpec((de, d2), lambda i: (0, 0)),
            pl.BlockSpec((1, d2), lambda i: (0, 0)),
            pl.BlockSpec((2, d2), lambda i: (0, 0)),
        ],
        out_specs=pl.BlockSpec((be_rows, d), lambda i: (i, 0)),
        out_shape=jax.ShapeDtypeStruct((n_edges, d), jnp.float32),
    )(m_arr, edge_feats, W_edge, b_edge.reshape(1, d2), params)

    s3 = functools.partial(
        pl.kernel,
        out_type=jax.ShapeDtypeStruct((NC, n_pad, d), jnp.float32),
        mesh=mesh,
        scratch_types=(
            [pltpu.VMEM((CB3,), jnp.int32)] * 5
            + [pltpu.VMEM((CB3, d), jnp.float32)] * 5
            + [pltpu.VMEM_SHARED((n_pad, d), jnp.float32)]
            + [pltpu.SemaphoreType.DMA] * 10
        ),
    )(functools.partial(_s3_body, n_pad, epw, nchunk3))
    partials = s3(act, dst)

    out = pl.pallas_call(
        _g4_body,
        grid=(1,),
        in_specs=[
            pl.BlockSpec((NC, n_nodes, d), lambda i: (0, 0, 0)),
            pl.BlockSpec((n_nodes, d), lambda i: (0, 0)),
            pl.BlockSpec((1, d), lambda i: (0, 0)),
            pl.BlockSpec((1, d), lambda i: (0, 0)),
        ],
        out_specs=pl.BlockSpec((n_nodes, d), lambda i: (0, 0)),
        out_shape=jax.ShapeDtypeStruct((n_nodes, d), jnp.float32),
    )(partials, node_feats, gamma_n.reshape(1, d), beta_n.reshape(1, d))
    return out
